# Initial kernel scaffold; baseline (speedup 1.0000x reference)
import functools

import jax
import jax.numpy as jnp
from jax import lax
from jax.experimental import pallas as pl
from jax.experimental.pallas import tpu as pltpu
from jax.experimental.pallas import tpu_sc as plsc

N = 10000
E = 320000
D_IN = 128
D_H = 256
D_OUT = 128

NP = 10240
NS = 16
CH = 80
RPW = E // NS // CH
RPS = NP // NS
BLK = 1024
GB = NP // BLK

_mesh = plsc.VectorSubcoreMesh(core_axis_name="c", subcore_axis_name="s")


def _deg_kernel(edges3d, ones_row, zeros16):
    @functools.partial(
        pl.kernel,
        out_type=jax.ShapeDtypeStruct((2 * NP, 16), jnp.float32),
        mesh=_mesh,
        scratch_types=[
            pltpu.VMEM((RPW, CH), jnp.int32),
            pltpu.VMEM((CH, 16), jnp.float32),
            pltpu.VMEM_SHARED((NP, 16), jnp.float32),
            pltpu.SemaphoreType.DMA,
        ],
    )
    def k(e_hbm, ones_hbm, z_hbm, deg_hbm, idx_v, ones_v, acc_sh, sem):
        c = lax.axis_index("c")
        s = lax.axis_index("s")
        pltpu.async_copy(e_hbm.at[c, pl.ds(s * RPW, RPW)], idx_v, sem).wait()
        pltpu.async_copy(ones_hbm, ones_v, sem).wait()
        pltpu.sync_copy(z_hbm, acc_sh.at[pl.ds(s * RPS, RPS)])
        plsc.subcore_barrier()

        @pl.loop(0, RPW)
        def _(i):
            pltpu.sync_copy(ones_v, acc_sh.at[idx_v.at[i]], add=True)

        plsc.subcore_barrier()
        pltpu.sync_copy(
            acc_sh.at[pl.ds(s * RPS, RPS)],
            deg_hbm.at[pl.ds(c * NP + s * RPS, RPS)],
        )

    return k(edges3d, ones_row, zeros16)


def _agg_kernel(y, src3d, dst3d, zeros, dh):
    @functools.partial(
        pl.kernel,
        out_type=jax.ShapeDtypeStruct((2 * NP, dh), jnp.float32),
        mesh=_mesh,
        scratch_types=[
            pltpu.VMEM((RPW, CH), jnp.int32),
            pltpu.VMEM((RPW, CH), jnp.int32),
            pltpu.VMEM((2, CH, dh), jnp.float32),
            pltpu.VMEM_SHARED((NP, dh), jnp.float32),
            pltpu.SemaphoreType.DMA,
            pltpu.SemaphoreType.DMA,
        ],
    )
    def k(y_hbm, src_hbm, dst_hbm, z_hbm, out_hbm, si_v, di_v, buf_v, acc_sh,
          sem0, sem1):
        c = lax.axis_index("c")
        s = lax.axis_index("s")
        pltpu.async_copy(src_hbm.at[c, pl.ds(s * RPW, RPW)], si_v, sem0).wait()
        pltpu.async_copy(dst_hbm.at[pl.ds(s * RPW, RPW)], di_v, sem0).wait()
        pltpu.sync_copy(z_hbm, acc_sh.at[pl.ds(s * RPS, RPS)])
        plsc.subcore_barrier()

        pltpu.async_copy(y_hbm.at[si_v.at[0]], buf_v.at[0], sem0)

        @pl.loop(0, RPW, step=2)
        def _(i):
            pltpu.async_copy(y_hbm.at[si_v.at[i + 1]], buf_v.at[1], sem1)
            pltpu.make_async_copy(y_hbm.at[si_v.at[i]], buf_v.at[0], sem0).wait()
            pltpu.sync_copy(buf_v.at[0], acc_sh.at[di_v.at[i]], add=True)

            @pl.when(i + 2 < RPW)
            def _():
                pltpu.async_copy(y_hbm.at[si_v.at[i + 2]], buf_v.at[0], sem0)

            pltpu.make_async_copy(y_hbm.at[si_v.at[i + 1]], buf_v.at[1],
                                  sem1).wait()
            pltpu.sync_copy(buf_v.at[1], acc_sh.at[di_v.at[i + 1]], add=True)

        plsc.subcore_barrier()
        pltpu.sync_copy(
            acc_sh.at[pl.ds(s * RPS, RPS)],
            out_hbm.at[pl.ds(c * NP + s * RPS, RPS)],
        )

    return k(y, src3d, dst3d, zeros)


def _mm1(x, w1):
    def body(x_ref, w_ref, o_ref):
        o_ref[...] = jnp.dot(x_ref[...], w_ref[...],
                             preferred_element_type=jnp.float32)

    return pl.pallas_call(
        body,
        grid=(GB,),
        in_specs=[
            pl.BlockSpec((BLK, D_IN), lambda i: (i, 0)),
            pl.BlockSpec((D_IN, D_H), lambda i: (0, 0)),
        ],
        out_specs=pl.BlockSpec((BLK, D_H), lambda i: (i, 0)),
        out_shape=jax.ShapeDtypeStruct((NP, D_H), jnp.float32),
    )(x, w1)


def _scale_split(z, deg):
    def body(z_ref, d_ref, o_ref):
        ns = lax.rsqrt(d_ref[:, :1] + 1.0)
        o_ref[...] = z_ref[...] * ns

    return pl.pallas_call(
        body,
        grid=(GB, 2),
        in_specs=[
            pl.BlockSpec((BLK, 128), lambda i, j: (i, j)),
            pl.BlockSpec((BLK, 16), lambda i, j: (i, 0)),
        ],
        out_specs=pl.BlockSpec((BLK, 128), lambda i, j: (i + GB * j, 0)),
        out_shape=jax.ShapeDtypeStruct((2 * NP, 128), jnp.float32),
    )(z, deg)


def _layer_mid(p, y, deg, w, b, dh_out):
    def body(pl_ref, pr_ref, yl_ref, yr_ref, ds_ref, dd_ref, w_ref, b_ref,
             o_ref):
        nd = lax.rsqrt(dd_ref[:, :1] + 1.0)
        ns = lax.rsqrt(ds_ref[:, :1] + 1.0)
        agg = jnp.concatenate(
            [pl_ref[...] + yl_ref[...], pr_ref[...] + yr_ref[...]], axis=1)
        h = jax.nn.relu(agg * nd + b_ref[...]) * ns
        o_ref[...] = jnp.dot(h, w_ref[...], preferred_element_type=jnp.float32)

    dh_in = y.shape[1]
    return pl.pallas_call(
        body,
        grid=(GB, 2),
        in_specs=[
            pl.BlockSpec((BLK, dh_in), lambda i, j: (i, 0)),
            pl.BlockSpec((BLK, dh_in), lambda i, j: (i + GB, 0)),
            pl.BlockSpec((BLK, dh_in), lambda i, j: (i, 0)),
            pl.BlockSpec((BLK, dh_in), lambda i, j: (i + GB, 0)),
            pl.BlockSpec((BLK, 16), lambda i, j: (i, 0)),
            pl.BlockSpec((BLK, 16), lambda i, j: (i + GB, 0)),
            pl.BlockSpec((D_H, dh_out), lambda i, j: (0, j)),
            pl.BlockSpec((1, D_H), lambda i, j: (0, 0)),
        ],
        out_specs=pl.BlockSpec((BLK, dh_out), lambda i, j: (i + GB * j, 0)),
        out_shape=jax.ShapeDtypeStruct((2 * NP, dh_out), jnp.float32),
    )(p, p, y, y, deg, deg, w, b)


def _layer_out(p, y, deg, b):
    def body(pl_ref, pr_ref, yl_ref, yr_ref, dd_ref, b_ref, o_ref):
        nd = lax.rsqrt(dd_ref[:, :1] + 1.0)
        agg = jnp.concatenate(
            [pl_ref[...] + yl_ref[...], pr_ref[...] + yr_ref[...]], axis=1)
        o_ref[...] = agg * nd + b_ref[...]

    dh = y.shape[1]
    return pl.pallas_call(
        body,
        grid=(GB,),
        in_specs=[
            pl.BlockSpec((BLK, dh), lambda i: (i, 0)),
            pl.BlockSpec((BLK, dh), lambda i: (i + GB, 0)),
            pl.BlockSpec((BLK, dh), lambda i: (i, 0)),
            pl.BlockSpec((BLK, dh), lambda i: (i + GB, 0)),
            pl.BlockSpec((BLK, 16), lambda i: (i + GB, 0)),
            pl.BlockSpec((1, D_OUT), lambda i: (0, 0)),
        ],
        out_specs=pl.BlockSpec((BLK, D_OUT), lambda i: (i, 0)),
        out_shape=jax.ShapeDtypeStruct((NP, D_OUT), jnp.float32),
    )(p, p, y, y, deg, b)


@jax.jit
def kernel(x, edge_index, W1, b1, W2, b2, W3, b3):
    src = edge_index[0]
    dst = edge_index[1]

    xp = jnp.zeros((NP, D_IN), jnp.float32).at[:N].set(x)
    edges3d = edge_index.reshape(2, E // CH, CH)
    src3d = jnp.stack([src, src + NP]).reshape(2, E // CH, CH)
    dst3d = dst.reshape(E // CH, CH)

    ones_row = jnp.ones((CH, 16), jnp.float32)
    zeros16 = jnp.zeros((RPS, 16), jnp.float32)
    zeros128 = jnp.zeros((RPS, 128), jnp.float32)
    zeros64 = jnp.zeros((RPS, 64), jnp.float32)

    b1r = b1.reshape(1, D_H)
    b2r = b2.reshape(1, D_H)
    b3r = b3.reshape(1, D_OUT)

    deg = _deg_kernel(edges3d, ones_row, zeros16)
    z1 = _mm1(xp, W1)

    y1 = _scale_split(z1, deg)
    p1 = _agg_kernel(y1, src3d, dst3d, zeros128, 128)
    y2 = _layer_mid(p1, y1, deg, W2, b1r, 128)
    p2 = _agg_kernel(y2, src3d, dst3d, zeros128, 128)
    y3 = _layer_mid(p2, y2, deg, W3, b2r, 64)
    p3 = _agg_kernel(y3, src3d, dst3d, zeros64, 64)
    out = _layer_out(p3, y3, deg, b3r)
    return out[:N]


# speedup vs baseline: 8.6805x; 8.6805x over previous
import functools

import jax
import jax.numpy as jnp
from jax import lax
from jax.experimental import pallas as pl
from jax.experimental.pallas import tpu as pltpu
from jax.experimental.pallas import tpu_sc as plsc

N = 10000
E = 320000
D_IN = 128
D_H = 256
D_OUT = 128

NP = 10240
HN = NP // 2
NS = 16
CH = 128
EP = 327680
RPW = EP // NS // CH
RPS = HN // NS
ZR = 80
BLK = 1024
GB = NP // BLK
IGN = 2**30


def _mesh():
    return plsc.VectorSubcoreMesh(core_axis_name="c", subcore_axis_name="s")


def _fill_zeros(z_v):
    @pl.loop(0, ZR)
    def _(r):
        @pl.loop(0, 128, step=16)
        def _(cc):
            z_v[r, pl.ds(cc, 16)] = jnp.zeros((16,), jnp.float32)


def _zero_acc(z_v, acc_sh, s):
    for t in range(RPS // ZR):
        pltpu.sync_copy(z_v, acc_sh.at[pl.ds(s * RPS + t * ZR, ZR)])


def _deg_kernel(dsidx, ddidx):
    @functools.partial(
        pl.kernel,
        out_type=jax.ShapeDtypeStruct((2 * NP, 128), jnp.float32),
        mesh=_mesh(),
        scratch_types=[
            pltpu.VMEM((RPW, CH), jnp.int32),
            pltpu.VMEM((CH, 128), jnp.float32),
            pltpu.VMEM((ZR, 128), jnp.float32),
            pltpu.VMEM_SHARED((HN, 128), jnp.float32),
            pltpu.SemaphoreType.DMA,
        ],
    )
    def k(ds_hbm, dd_hbm, deg_hbm, idx_v, ones_v, z_v, acc_sh, sem):
        c = lax.axis_index("c")
        s = lax.axis_index("s")
        _fill_zeros(z_v)

        @pl.loop(0, CH)
        def _(r):
            @pl.loop(0, 128, step=16)
            def _(cc):
                ones_v[r, pl.ds(cc, 16)] = jnp.ones((16,), jnp.float32)

        for half, src_hbm in ((0, ds_hbm), (1, dd_hbm)):
            pltpu.async_copy(src_hbm.at[c, pl.ds(s * RPW, RPW)], idx_v,
                             sem).wait()
            _zero_acc(z_v, acc_sh, s)
            plsc.subcore_barrier()

            @pl.loop(0, RPW)
            def _(i):
                gi = plsc.Indices(idx_v.at[i], ignored_value=IGN)
                pltpu.sync_copy(ones_v, acc_sh.at[gi], add=True)

            plsc.subcore_barrier()
            pltpu.sync_copy(
                acc_sh.at[pl.ds(s * RPS, RPS)],
                deg_hbm.at[pl.ds(half * NP + c * HN + s * RPS, RPS)],
            )
            plsc.subcore_barrier()

    return k(dsidx, ddidx)


def _agg_kernel(y, gidx, sidx, nh):
    @functools.partial(
        pl.kernel,
        out_type=jax.ShapeDtypeStruct((nh * NP, 128), jnp.float32),
        mesh=_mesh(),
        scratch_types=[
            pltpu.VMEM((RPW, CH), jnp.int32),
            pltpu.VMEM((RPW, CH), jnp.int32),
            pltpu.VMEM((2, CH, 128), jnp.float32),
            pltpu.VMEM((ZR, 128), jnp.float32),
            pltpu.VMEM_SHARED((HN, 128), jnp.float32),
            pltpu.SemaphoreType.DMA,
            pltpu.SemaphoreType.DMA,
        ],
    )
    def k(y_hbm, g_hbm, d_hbm, out_hbm, si_v, di_v, buf_v, z_v, acc_sh,
          sem0, sem1):
        c = lax.axis_index("c")
        s = lax.axis_index("s")
        _fill_zeros(z_v)
        pltpu.async_copy(d_hbm.at[c, pl.ds(s * RPW, RPW)], di_v, sem0).wait()

        for h in range(nh):
            pltpu.async_copy(g_hbm.at[c, h, pl.ds(s * RPW, RPW)], si_v,
                             sem0).wait()
            _zero_acc(z_v, acc_sh, s)
            plsc.subcore_barrier()

            def gi(i):
                return plsc.Indices(si_v.at[i], ignored_value=IGN)

            def di(i):
                return plsc.Indices(di_v.at[i], ignored_value=IGN)

            pltpu.async_copy(y_hbm.at[gi(0)], buf_v.at[0], sem0)

            @pl.loop(0, RPW, step=2)
            def _(i):
                pltpu.async_copy(y_hbm.at[gi(i + 1)], buf_v.at[1], sem1)
                pltpu.make_async_copy(y_hbm.at[gi(i)], buf_v.at[0],
                                      sem0).wait()
                pltpu.sync_copy(buf_v.at[0], acc_sh.at[di(i)], add=True)

                @pl.when(i + 2 < RPW)
                def _():
                    pltpu.async_copy(y_hbm.at[gi(i + 2)], buf_v.at[0], sem0)

                pltpu.make_async_copy(y_hbm.at[gi(i + 1)], buf_v.at[1],
                                      sem1).wait()
                pltpu.sync_copy(buf_v.at[1], acc_sh.at[di(i + 1)], add=True)

            plsc.subcore_barrier()
            pltpu.sync_copy(
                acc_sh.at[pl.ds(s * RPS, RPS)],
                out_hbm.at[pl.ds(h * NP + c * HN + s * RPS, RPS)],
            )
            plsc.subcore_barrier()

    return k(y, gidx, sidx)


def _mm1(x, w1s):
    def body(x_ref, w_ref, o_ref):
        o_ref[...] = jnp.dot(x_ref[...], w_ref[0],
                             preferred_element_type=jnp.float32)

    return pl.pallas_call(
        body,
        grid=(GB, 2),
        in_specs=[
            pl.BlockSpec((BLK, D_IN), lambda i, j: (i, 0)),
            pl.BlockSpec((1, D_IN, 128), lambda i, j: (j, 0, 0)),
        ],
        out_specs=pl.BlockSpec((BLK, 128), lambda i, j: (i + GB * j, 0)),
        out_shape=jax.ShapeDtypeStruct((2 * NP, 128), jnp.float32),
    )(x, w1s)


def _scale_split(z, deg):
    def body(z_ref, d_ref, o_ref):
        ns = lax.rsqrt(d_ref[:, :1] + 1.0)
        o_ref[...] = z_ref[...] * ns

    return pl.pallas_call(
        body,
        grid=(GB, 2),
        in_specs=[
            pl.BlockSpec((BLK, 128), lambda i, j: (i + GB * j, 0)),
            pl.BlockSpec((BLK, 128), lambda i, j: (i, 0)),
        ],
        out_specs=pl.BlockSpec((BLK, 128), lambda i, j: (i + GB * j, 0)),
        out_shape=jax.ShapeDtypeStruct((2 * NP, 128), jnp.float32),
    )(z, deg)


def _layer_mid(p, y, deg, w, b):
    def body(pl_ref, pr_ref, yl_ref, yr_ref, ds_ref, dd_ref, w_ref, b_ref,
             o_ref):
        nd = lax.rsqrt(dd_ref[:, :1] + 1.0)
        ns = lax.rsqrt(ds_ref[:, :1] + 1.0)
        agg = jnp.concatenate(
            [pl_ref[...] + yl_ref[...], pr_ref[...] + yr_ref[...]], axis=1)
        h = jax.nn.relu(agg * nd + b_ref[...]) * ns
        o_ref[...] = jnp.dot(h, w_ref[0], preferred_element_type=jnp.float32)

    nh_out = w.shape[0]
    return pl.pallas_call(
        body,
        grid=(GB, nh_out),
        in_specs=[
            pl.BlockSpec((BLK, 128), lambda i, j: (i, 0)),
            pl.BlockSpec((BLK, 128), lambda i, j: (i + GB, 0)),
            pl.BlockSpec((BLK, 128), lambda i, j: (i, 0)),
            pl.BlockSpec((BLK, 128), lambda i, j: (i + GB, 0)),
            pl.BlockSpec((BLK, 128), lambda i, j: (i, 0)),
            pl.BlockSpec((BLK, 128), lambda i, j: (i + GB, 0)),
            pl.BlockSpec((1, D_H, 128), lambda i, j: (j, 0, 0)),
            pl.BlockSpec((1, D_H), lambda i, j: (0, 0)),
        ],
        out_specs=pl.BlockSpec((BLK, 128), lambda i, j: (i + GB * j, 0)),
        out_shape=jax.ShapeDtypeStruct((nh_out * NP, 128), jnp.float32),
    )(p, p, y, y, deg, deg, w, b)


def _layer_out(p, y, deg, b):
    def body(p_ref, y_ref, dd_ref, b_ref, o_ref):
        nd = lax.rsqrt(dd_ref[:, :1] + 1.0)
        o_ref[...] = (p_ref[...] + y_ref[...]) * nd + b_ref[...]

    return pl.pallas_call(
        body,
        grid=(GB,),
        in_specs=[
            pl.BlockSpec((BLK, 128), lambda i: (i, 0)),
            pl.BlockSpec((BLK, 128), lambda i: (i, 0)),
            pl.BlockSpec((BLK, 128), lambda i: (i + GB, 0)),
            pl.BlockSpec((1, D_OUT), lambda i: (0, 0)),
        ],
        out_specs=pl.BlockSpec((BLK, D_OUT), lambda i: (i, 0)),
        out_shape=jax.ShapeDtypeStruct((NP, D_OUT), jnp.float32),
    )(p, y, deg, b)


@jax.jit
def kernel(x, edge_index, W1, b1, W2, b2, W3, b3):
    src = edge_index[0]
    dst = edge_index[1]

    xp = jnp.zeros((NP, D_IN), jnp.float32).at[:N].set(x)
    pad = jnp.full((EP - E,), -1, jnp.int32)
    srcp = jnp.concatenate([src, pad])
    dstp = jnp.concatenate([dst, pad])

    gidx = []
    sidx = []
    dsidx = []
    for c in range(2):
        in_dst = (dstp >= c * HN) & (dstp < (c + 1) * HN)
        in_src = (srcp >= c * HN) & (srcp < (c + 1) * HN)
        gidx.append([jnp.where(in_dst, srcp + h * NP, IGN) for h in range(2)])
        sidx.append(jnp.where(in_dst, dstp - c * HN, IGN))
        dsidx.append(jnp.where(in_src, srcp - c * HN, IGN))
    gidx = jnp.stack([jnp.stack(g) for g in gidx])
    sidx = jnp.stack(sidx)
    dsidx = jnp.stack(dsidx)
    gidx = gidx.reshape(2, 2, EP // CH, CH)
    sidx = sidx.reshape(2, EP // CH, CH)
    dsidx = dsidx.reshape(2, EP // CH, CH)

    b1r = b1.reshape(1, D_H)
    b2r = b2.reshape(1, D_H)
    b3r = b3.reshape(1, D_OUT)
    w1s = W1.reshape(D_IN, 2, 128).transpose(1, 0, 2)
    w2s = W2.reshape(D_H, 2, 128).transpose(1, 0, 2)
    w3s = W3.reshape(D_H, 1, 128).transpose(1, 0, 2)

    deg = _deg_kernel(dsidx, sidx)
    z1 = _mm1(xp, w1s)

    y1 = _scale_split(z1, deg)
    p1 = _agg_kernel(y1, gidx, sidx, 2)
    y2 = _layer_mid(p1, y1, deg, w2s, b1r)
    p2 = _agg_kernel(y2, gidx, sidx, 2)
    y3 = _layer_mid(p2, y2, deg, w3s, b2r)
    p3 = _agg_kernel(y3, gidx[:, :1], sidx, 1)
    out = _layer_out(p3, y3, deg, b3r)
    return out[:N]
